# baseline (speedup 1.0000x reference)
import jax
import jax.numpy as jnp
from jax.experimental import pallas as pl
from jax.experimental.pallas import tpu as pltpu

_N_SLOTS = 64
_EMBED_DIM = 256
_N_CODES = 8192
_BETA = 0.25
_CHUNK = 1024
_N_CHUNKS = _N_CODES // _CHUNK


def _fused_vq_kernel(slots_ref, W_enc_ref, b_enc_ref, W_prev_ref, b_prev_ref,
                     cb_hbm, W_post_hbm, b_post_ref, W_dec_hbm, b_dec_ref,
                     rec_ref, loss_ref, idx_ref,
                     cb_vmem, wpost_vmem, wdec_vmem, sems):
    f32 = jnp.float32
    C = _CHUNK
    chunk_cps = [
        pltpu.make_async_copy(cb_hbm.at[pl.ds(k * C, C), :],
                              cb_vmem.at[pl.ds(k * C, C), :],
                              sems.at[k])
        for k in range(_N_CHUNKS)
    ]
    wpost_cp = pltpu.make_async_copy(W_post_hbm, wpost_vmem,
                                     sems.at[_N_CHUNKS])
    wdec_cp = pltpu.make_async_copy(W_dec_hbm, wdec_vmem,
                                    sems.at[_N_CHUNKS + 1])
    chunk_cps[0].start()
    chunk_cps[1].start()

    h = jnp.maximum(
        jnp.dot(slots_ref[...], W_enc_ref[...], preferred_element_type=f32)
        + b_enc_ref[...], 0.0)
    s = (jnp.dot(h, W_prev_ref[...], preferred_element_type=f32)
         + b_prev_ref[...])
    a = jnp.sum(s * s, axis=1, keepdims=True)

    big = jnp.int32(jnp.iinfo(jnp.int32).max)
    run_d = run_i = run_z = None
    for k in range(_N_CHUNKS):
        chunk_cps[k].wait()
        if k + 2 < _N_CHUNKS:
            chunk_cps[k + 2].start()
        elif k + 2 == _N_CHUNKS:
            wpost_cp.start()
            wdec_cp.start()
        cb = cb_vmem[pl.ds(k * C, C), :]
        m = jax.lax.dot_general(s, cb, (((1,), (1,)), ((), ())),
                                preferred_element_type=f32)
        cn = jnp.sum(cb * cb, axis=1)
        d = (a - 2.0 * m) + cn[None, :]
        dmin = jnp.min(d, axis=1, keepdims=True)
        col = jax.lax.broadcasted_iota(jnp.int32, d.shape, 1) + k * C
        lidx = jnp.min(jnp.where(d == dmin, col, big), axis=1)[:, None]
        onehot = (col == lidx).astype(f32)
        lzq = jnp.dot(onehot, cb, preferred_element_type=f32)
        if k == 0:
            run_d, run_i, run_z = dmin, lidx, lzq
        else:
            better = dmin < run_d
            run_d = jnp.where(better, dmin, run_d)
            run_i = jnp.where(better, lidx, run_i)
            run_z = jnp.where(better, lzq, run_z)

    diff = run_z - s
    loss = (1.0 + _BETA) * jnp.sum(diff * diff) / (_N_SLOTS * _EMBED_DIM)
    loss_ref[...] = jnp.reshape(loss, (1, 1))
    wpost_cp.wait()
    dec_in = (jnp.dot(run_z, wpost_vmem[...], preferred_element_type=f32)
              + b_post_ref[...])
    wdec_cp.wait()
    rec = (jnp.dot(dec_in, wdec_vmem[...], preferred_element_type=f32)
           + b_dec_ref[...])
    rec_ref[...] = jnp.clip(rec, -1.0, 1.0)
    idx_ref[...] = run_i


def kernel(img, targets, slots, W_enc, b_enc, W_prev, b_prev, codebook,
           W_post, b_post, W_dec, b_dec):
    bs = img.shape[0]
    enc_dim = W_dec.shape[1]
    f32 = jnp.float32
    vmem = pl.BlockSpec(memory_space=pltpu.MemorySpace.VMEM)
    hbm = pl.BlockSpec(memory_space=pltpu.MemorySpace.HBM)
    rec1, loss, idx = pl.pallas_call(
        _fused_vq_kernel,
        in_specs=[vmem, vmem, vmem, vmem, vmem,
                  hbm, hbm, vmem, hbm, vmem],
        out_specs=[vmem, vmem, vmem],
        out_shape=[
            jax.ShapeDtypeStruct((_N_SLOTS, enc_dim), f32),
            jax.ShapeDtypeStruct((1, 1), f32),
            jax.ShapeDtypeStruct((_N_SLOTS, 1), jnp.int32),
        ],
        scratch_shapes=[
            pltpu.VMEM((_N_CODES, _EMBED_DIM), f32),
            pltpu.VMEM(W_post.shape, f32),
            pltpu.VMEM(W_dec.shape, f32),
            pltpu.SemaphoreType.DMA((_N_CHUNKS + 2,)),
        ],
    )(slots, W_enc, b_enc.reshape(1, -1), W_prev, b_prev.reshape(1, -1),
      codebook, W_post, b_post.reshape(1, -1), W_dec, b_dec.reshape(1, -1))
    rec = jnp.broadcast_to(rec1[None], (bs, _N_SLOTS, enc_dim))
    q_indices = jnp.broadcast_to(idx.reshape(1, _N_SLOTS), (bs, _N_SLOTS))
    return rec, jnp.reshape(loss, ()), q_indices


# speedup vs baseline: 1.0243x; 1.0243x over previous
import jax
import jax.numpy as jnp
from jax.experimental import pallas as pl
from jax.experimental.pallas import tpu as pltpu

_N_SLOTS = 64
_EMBED_DIM = 256
_N_CODES = 8192
_BETA = 0.25
_CHUNK = 1024
_N_CHUNKS = _N_CODES // _CHUNK


def _fused_vq_kernel(slots_ref, W_enc_ref, b_enc_ref, W_prev_ref, b_prev_ref,
                     cb_hbm, W_post_hbm, b_post_ref, W_dec_hbm, b_dec_ref,
                     rec_ref, loss_ref, idx_ref,
                     cb_vmem, d_vmem, wpost_vmem, wdec_vmem, sems):
    f32 = jnp.float32
    C = _CHUNK
    chunk_cps = [
        pltpu.make_async_copy(cb_hbm.at[pl.ds(k * C, C), :],
                              cb_vmem.at[pl.ds(k * C, C), :],
                              sems.at[k])
        for k in range(_N_CHUNKS)
    ]
    wpost_cp = pltpu.make_async_copy(W_post_hbm, wpost_vmem,
                                     sems.at[_N_CHUNKS])
    wdec_cp = pltpu.make_async_copy(W_dec_hbm, wdec_vmem,
                                    sems.at[_N_CHUNKS + 1])
    chunk_cps[0].start()
    chunk_cps[1].start()

    h = jnp.maximum(
        jnp.dot(slots_ref[...], W_enc_ref[...], preferred_element_type=f32)
        + b_enc_ref[...], 0.0)
    s = (jnp.dot(h, W_prev_ref[...], preferred_element_type=f32)
         + b_prev_ref[...])
    a = jnp.sum(s * s, axis=1, keepdims=True)

    run_min = None
    for k in range(_N_CHUNKS):
        chunk_cps[k].wait()
        if k + 2 < _N_CHUNKS:
            chunk_cps[k + 2].start()
        elif k + 2 == _N_CHUNKS:
            wpost_cp.start()
            wdec_cp.start()
        cb = cb_vmem[pl.ds(k * C, C), :]
        m = jax.lax.dot_general(s, cb, (((1,), (1,)), ((), ())),
                                preferred_element_type=f32)
        cn = jnp.sum(cb * cb, axis=1)
        d = (a - 2.0 * m) + cn[None, :]
        d_vmem[:, pl.ds(k * C, C)] = d
        run_min = d if k == 0 else jnp.minimum(run_min, d)

    dmin = jnp.min(run_min, axis=1, keepdims=True)
    dfull = d_vmem[...]
    col = jax.lax.broadcasted_iota(jnp.int32, dfull.shape, 1)
    big = jnp.int32(jnp.iinfo(jnp.int32).max)
    idx = jnp.min(jnp.where(dfull == dmin, col, big), axis=1)[:, None]
    onehot = (col == idx).astype(f32)
    zq = jnp.dot(onehot, cb_vmem[...], preferred_element_type=f32)

    diff = zq - s
    loss = (1.0 + _BETA) * jnp.sum(diff * diff) / (_N_SLOTS * _EMBED_DIM)
    loss_ref[...] = jnp.reshape(loss, (1, 1))
    wpost_cp.wait()
    dec_in = (jnp.dot(zq, wpost_vmem[...], preferred_element_type=f32)
              + b_post_ref[...])
    wdec_cp.wait()
    rec = (jnp.dot(dec_in, wdec_vmem[...], preferred_element_type=f32)
           + b_dec_ref[...])
    rec_ref[...] = jnp.clip(rec, -1.0, 1.0)
    idx_ref[...] = idx


def kernel(img, targets, slots, W_enc, b_enc, W_prev, b_prev, codebook,
           W_post, b_post, W_dec, b_dec):
    bs = img.shape[0]
    enc_dim = W_dec.shape[1]
    f32 = jnp.float32
    vmem = pl.BlockSpec(memory_space=pltpu.MemorySpace.VMEM)
    hbm = pl.BlockSpec(memory_space=pltpu.MemorySpace.HBM)
    rec1, loss, idx = pl.pallas_call(
        _fused_vq_kernel,
        in_specs=[vmem, vmem, vmem, vmem, vmem,
                  hbm, hbm, vmem, hbm, vmem],
        out_specs=[vmem, vmem, vmem],
        out_shape=[
            jax.ShapeDtypeStruct((_N_SLOTS, enc_dim), f32),
            jax.ShapeDtypeStruct((1, 1), f32),
            jax.ShapeDtypeStruct((_N_SLOTS, 1), jnp.int32),
        ],
        scratch_shapes=[
            pltpu.VMEM((_N_CODES, _EMBED_DIM), f32),
            pltpu.VMEM((_N_SLOTS, _N_CODES), f32),
            pltpu.VMEM(W_post.shape, f32),
            pltpu.VMEM(W_dec.shape, f32),
            pltpu.SemaphoreType.DMA((_N_CHUNKS + 2,)),
        ],
    )(slots, W_enc, b_enc.reshape(1, -1), W_prev, b_prev.reshape(1, -1),
      codebook, W_post, b_post.reshape(1, -1), W_dec, b_dec.reshape(1, -1))
    rec = jnp.broadcast_to(rec1[None], (bs, _N_SLOTS, enc_dim))
    q_indices = jnp.broadcast_to(idx.reshape(1, _N_SLOTS), (bs, _N_SLOTS))
    return rec, jnp.reshape(loss, ()), q_indices


# speedup vs baseline: 1.4800x; 1.4449x over previous
import jax
import jax.numpy as jnp
from jax.experimental import pallas as pl

_N_SLOTS = 64
_EMBED_DIM = 256
_N_CODES = 8192
_BETA = 0.25
_BATCH = 8


def _fused_vq_kernel(slots_ref, W_enc_ref, b_enc_ref, W_prev_ref, b_prev_ref,
                     cb_ref, W_post_ref, b_post_ref, W_dec_ref, b_dec_ref,
                     rec_ref, loss_ref, idx_ref):
    f32 = jnp.float32
    h = jnp.maximum(
        jnp.dot(slots_ref[...], W_enc_ref[...], preferred_element_type=f32)
        + b_enc_ref[...], 0.0)
    s = (jnp.dot(h, W_prev_ref[...], preferred_element_type=f32)
         + b_prev_ref[...])
    cb = cb_ref[...]
    a = jnp.sum(s * s, axis=1, keepdims=True)
    m = jax.lax.dot_general(s, cb, (((1,), (1,)), ((), ())),
                            preferred_element_type=f32)
    cn = jnp.sum(cb * cb, axis=1)
    d = (a - 2.0 * m) + cn[None, :]
    dmin = jnp.min(d, axis=1, keepdims=True)
    col = jax.lax.broadcasted_iota(jnp.int32, d.shape, 1)
    big = jnp.int32(jnp.iinfo(jnp.int32).max)
    idx = jnp.min(jnp.where(d == dmin, col, big), axis=1)
    onehot = (col == idx[:, None]).astype(f32)
    zq = jnp.dot(onehot, cb, preferred_element_type=f32)
    diff = zq - s
    loss = (1.0 + _BETA) * jnp.sum(diff * diff) / (_N_SLOTS * _EMBED_DIM)
    loss_ref[...] = jnp.reshape(loss, (1, 1))
    dec_in = (jnp.dot(zq, W_post_ref[...], preferred_element_type=f32)
              + b_post_ref[...])
    rec = (jnp.dot(dec_in, W_dec_ref[...], preferred_element_type=f32)
           + b_dec_ref[...])
    rec = jnp.clip(rec, -1.0, 1.0)
    for b in range(_BATCH):
        rec_ref[b] = rec
    idx_ref[...] = jnp.broadcast_to(idx[None, :], (_BATCH, _N_SLOTS))


def kernel(img, targets, slots, W_enc, b_enc, W_prev, b_prev, codebook,
           W_post, b_post, W_dec, b_dec):
    bs = img.shape[0]
    enc_dim = W_dec.shape[1]
    rec, loss, idx = pl.pallas_call(
        _fused_vq_kernel,
        out_shape=[
            jax.ShapeDtypeStruct((bs, _N_SLOTS, enc_dim), jnp.float32),
            jax.ShapeDtypeStruct((1, 1), jnp.float32),
            jax.ShapeDtypeStruct((bs, _N_SLOTS), jnp.int32),
        ],
    )(slots, W_enc, b_enc.reshape(1, -1), W_prev, b_prev.reshape(1, -1),
      codebook, W_post, b_post.reshape(1, -1), W_dec, b_dec.reshape(1, -1))
    return rec, jnp.reshape(loss, ()), idx
